# baseline (speedup 1.0000x reference)
import functools

import numpy as np

import jax
import jax.numpy as jnp
from jax import lax
from jax.experimental import pallas as pl
from jax.experimental.pallas import tpu as pltpu
from jax.experimental.pallas import tpu_sc as plsc

EMBED = 256
HEADS = 8
LEVELS = 4
POINTS = 4
SHAPES = ((64, 64), (32, 32), (16, 16), (8, 8))
BS = 2
NQ = sum(h * w for h, w in SHAPES)
HEAD_DIM = EMBED // HEADS
HLP = HEADS * LEVELS * POINTS
TQ = 680
NJ = NQ // TQ

_LMAP = np.array([(c // POINTS) % LEVELS for c in range(HLP)], np.int32)
_HMAP = np.array([c // (LEVELS * POINTS) for c in range(HLP)], np.int32)
_STARTS = np.cumsum([0] + [h * w for h, w in SHAPES])[:LEVELS].astype(np.int32)
_WCOL = np.array([SHAPES[l][1] for l in _LMAP], np.int32)
_HCOL = np.array([SHAPES[l][0] for l in _LMAP], np.int32)
_STARTCOL = _STARTS[_LMAP]

N_ITEMS = BS * NQ
N_TILES = 32
ITEMS_PER_TILE = N_ITEMS // N_TILES


def _prep_body(q_ref, rpx_ref, rpy_ref, wval_ref, bval_ref, woffx_ref,
               boffx_ref, woffy_ref, boffy_ref, wattn_ref, battn_ref,
               ci_ref, cf_ref, val_ref, idx_ref, w_ref):
    b = pl.program_id(0)
    q = q_ref[0]

    dot = functools.partial(
        lax.dot_general, dimension_numbers=(((1,), (1,)), ((), ())),
        preferred_element_type=jnp.float32)

    val_ref[0] = dot(q, wval_ref[...]) + bval_ref[0][None, :]

    offx = dot(q, woffx_ref[...]) + boffx_ref[0][None, :]
    offy = dot(q, woffy_ref[...]) + boffy_ref[0][None, :]
    logits = dot(q, wattn_ref[...]) + battn_ref[0][None, :]

    z = logits.reshape(TQ, HEADS, LEVELS * POINTS)
    z = z - jnp.max(z, axis=-1, keepdims=True)
    e = jnp.exp(z)
    aw = (e / jnp.sum(e, axis=-1, keepdims=True)).reshape(TQ, HLP)

    wcol = cf_ref[0][None, :]
    hcol = cf_ref[1][None, :]
    wm1 = ci_ref[0][None, :]
    hm1 = ci_ref[1][None, :]
    start = ci_ref[2][None, :]
    hmap = ci_ref[3][None, :]
    wint = ci_ref[4][None, :]

    px = rpx_ref[0] * wcol + offx - 0.5
    py = rpy_ref[0] * hcol + offy - 0.5
    x0f = jnp.floor(px)
    y0f = jnp.floor(py)
    fx = px - x0f
    fy = py - y0f
    ix0 = x0f.astype(jnp.int32)
    iy0 = y0f.astype(jnp.int32)
    wxs = (1.0 - fx, fx)
    wys = (1.0 - fy, fy)

    base_row = b * (NQ * HEADS) + hmap
    for cy in range(2):
        for cx in range(2):
            ci = cy * 2 + cx
            ix = ix0 + cx
            iy = iy0 + cy
            valid = ((ix >= 0) & (ix <= wm1) & (iy >= 0) & (iy <= hm1))
            ixc = jnp.clip(ix, 0, wm1)
            iyc = jnp.clip(iy, 0, hm1)
            cell = start + iyc * wint + ixc
            idx_ref[0, :, ci, :] = base_row + cell * HEADS
            w_ref[0, :, ci, :] = aw * wxs[cx] * wys[cy] * valid.astype(jnp.float32)


def _out_body(m_ref, q_ref, wout_ref, bout_ref, o_ref):
    dot = functools.partial(
        lax.dot_general, dimension_numbers=(((1,), (1,)), ((), ())),
        preferred_element_type=jnp.float32)
    o_ref[0] = dot(m_ref[0], wout_ref[...]) + bout_ref[0][None, :] + q_ref[0]


def _sc_body(table_ref, idx_hbm, w_hbm, out_hbm, idx_v, w_v, rows_v, out_v,
             g_sem, io_sem, o_sem):
    nc = 2
    wid = lax.axis_index("s") * nc + lax.axis_index("c")
    base = wid * ITEMS_PER_TILE

    def fire_io(j, bb):
        pltpu.async_copy(idx_hbm.at[base + j],
                         idx_v.at[pl.ds(bb * 4, 4)], io_sem.at[bb])
        pltpu.async_copy(w_hbm.at[base + j], w_v.at[bb], io_sem.at[bb])

    def wait_io(bb):
        pltpu.make_async_copy(idx_hbm.at[0], idx_v.at[pl.ds(bb * 4, 4)],
                              io_sem.at[bb]).wait()
        pltpu.make_async_copy(w_hbm.at[0], w_v.at[bb], io_sem.at[bb]).wait()

    def fire_gathers(bb):
        for c in range(4):
            pltpu.async_copy(table_ref.at[idx_v.at[bb * 4 + c]],
                             rows_v.at[bb, c], g_sem.at[bb, c])

    def wait_gathers(bb):
        for c in range(4):
            pltpu.make_async_copy(table_ref.at[idx_v.at[bb * 4 + c]],
                                  rows_v.at[bb, c], g_sem.at[bb, c]).wait()

    def wait_out(bb):
        pltpu.make_async_copy(out_v.at[bb], out_hbm.at[0], o_sem.at[bb]).wait()

    fire_io(0, 0)
    fire_io(1, 1)
    wait_io(0)
    fire_gathers(0)

    def pair_body(i, carry):
        j0 = i * 2
        for b in range(2):
            j = j0 + b
            wait_gathers(b)

            @pl.when(j + 1 < ITEMS_PER_TILE)
            def _(b=b):
                wait_io(1 - b)
                fire_gathers(1 - b)

            @pl.when(j >= 2)
            def _(b=b):
                wait_out(b)

            w_slice = w_v.at[b]
            zero = jnp.zeros((16,), jnp.float32)
            for h in range(HEADS):
                parts = []
                for c in range(4):
                    rbase = c * HLP + h * 16

                    def k_body(k0, carry, b=b, c=c, h=h, rbase=rbase):
                        a0e, a0o, a1e, a1o = carry
                        for dk in range(8):
                            kk = k0 * 8 + dk
                            wv = plsc.load_gather(
                                w_slice,
                                [jnp.full((16,), rbase + kk, jnp.int32)])
                            r0 = rows_v[b, c, h * 16 + kk, pl.ds(0, 16)]
                            r1 = rows_v[b, c, h * 16 + kk, pl.ds(16, 16)]
                            if dk % 2 == 0:
                                a0e = a0e + wv * r0
                                a1e = a1e + wv * r1
                            else:
                                a0o = a0o + wv * r0
                                a1o = a1o + wv * r1
                        return a0e, a0o, a1e, a1o

                    a0e, a0o, a1e, a1o = lax.fori_loop(
                        0, 2, k_body, (zero, zero, zero, zero))
                    parts.append((a0e + a0o, a1e + a1o))
                acc0 = (parts[0][0] + parts[1][0]) + (parts[2][0] + parts[3][0])
                acc1 = (parts[0][1] + parts[1][1]) + (parts[2][1] + parts[3][1])
                out_v[b, pl.ds(h * 32, 16)] = acc0
                out_v[b, pl.ds(h * 32 + 16, 16)] = acc1
            pltpu.async_copy(out_v.at[b], out_hbm.at[base + j], o_sem.at[b])

            @pl.when(j + 2 < ITEMS_PER_TILE)
            def _(b=b, j=j):
                fire_io(j + 2, b)
        return carry

    lax.fori_loop(0, ITEMS_PER_TILE // 2, pair_body, 0)
    wait_out(0)
    wait_out(1)


def _run_prep(q_t, rpx, rpy, W_val, b_val, W_off_x, b_off_x, W_off_y,
              b_off_y, W_attn, b_attn, ci, cf, interpret=False):
    grid = (BS, NJ)
    full = lambda shape: pl.BlockSpec(shape, lambda b, j: tuple(0 for _ in shape))
    return pl.pallas_call(
        _prep_body,
        grid=grid,
        in_specs=[
            pl.BlockSpec((1, TQ, EMBED), lambda b, j: (b, j, 0)),
            pl.BlockSpec((1, TQ, HLP), lambda b, j: (b, j, 0)),
            pl.BlockSpec((1, TQ, HLP), lambda b, j: (b, j, 0)),
            full((EMBED, EMBED)),
            full((1, EMBED)),
            full((HLP, EMBED)),
            full((1, HLP)),
            full((HLP, EMBED)),
            full((1, HLP)),
            full((HLP, EMBED)),
            full((1, HLP)),
            full((8, HLP)),
            full((8, HLP)),
        ],
        out_specs=[
            pl.BlockSpec((1, TQ, EMBED), lambda b, j: (b, j, 0)),
            pl.BlockSpec((1, TQ, 4, HLP), lambda b, j: (b, j, 0, 0)),
            pl.BlockSpec((1, TQ, 4, HLP), lambda b, j: (b, j, 0, 0)),
        ],
        out_shape=[
            jax.ShapeDtypeStruct((BS, NQ, EMBED), jnp.float32),
            jax.ShapeDtypeStruct((BS, NQ, 4, HLP), jnp.int32),
            jax.ShapeDtypeStruct((BS, NQ, 4, HLP), jnp.float32),
        ],
        interpret=interpret,
    )(q_t, rpx, rpy, W_val, b_val, W_off_x, b_off_x, W_off_y, b_off_y,
      W_attn, b_attn, ci, cf)


def _run_out(msda, q_t, W_out, b_out, interpret=False):
    full = lambda shape: pl.BlockSpec(shape, lambda b, j: tuple(0 for _ in shape))
    return pl.pallas_call(
        _out_body,
        grid=(BS, NJ),
        in_specs=[
            pl.BlockSpec((1, TQ, EMBED), lambda b, j: (b, j, 0)),
            pl.BlockSpec((1, TQ, EMBED), lambda b, j: (b, j, 0)),
            full((EMBED, EMBED)),
            full((1, EMBED)),
        ],
        out_specs=pl.BlockSpec((1, TQ, EMBED), lambda b, j: (b, j, 0)),
        out_shape=jax.ShapeDtypeStruct((BS, NQ, EMBED), jnp.float32),
        interpret=interpret,
    )(msda, q_t, W_out, b_out)


def _run_sc(table, idx, w):
    mesh = plsc.VectorSubcoreMesh(core_axis_name="c", subcore_axis_name="s")
    kern = functools.partial(
        pl.kernel,
        mesh=mesh,
        out_type=jax.ShapeDtypeStruct((N_ITEMS, EMBED), jnp.float32),
        scratch_types=[
            pltpu.VMEM((8, HLP), jnp.int32),
            pltpu.VMEM((2, 4 * HLP), jnp.float32),
            pltpu.VMEM((2, 4, HLP, HEAD_DIM), jnp.float32),
            pltpu.VMEM((2, EMBED), jnp.float32),
            pltpu.SemaphoreType.DMA((2, 4)),
            pltpu.SemaphoreType.DMA((2,)),
            pltpu.SemaphoreType.DMA((2,)),
        ],
        compiler_params=pltpu.CompilerParams(
            needs_layout_passes=False, use_tc_tiling_on_sc=False),
    )(_sc_body)
    return kern(table, idx, w)


def kernel(query, reference_points, spatial_shapes, W_off, b_off, W_attn,
           b_attn, W_val, b_val, W_out, b_out):
    del spatial_shapes
    q_t = jnp.transpose(query, (1, 0, 2))

    W_off_r = W_off.reshape(HEADS, LEVELS, POINTS, 2, EMBED)
    b_off_r = b_off.reshape(HEADS, LEVELS, POINTS, 2)
    W_off_x = W_off_r[..., 0, :].reshape(HLP, EMBED)
    W_off_y = W_off_r[..., 1, :].reshape(HLP, EMBED)
    b_off_x = b_off_r[..., 0].reshape(1, HLP)
    b_off_y = b_off_r[..., 1].reshape(1, HLP)

    lmap = jnp.asarray(_LMAP)
    rpx = reference_points[..., 0][:, :, lmap]
    rpy = reference_points[..., 1][:, :, lmap]

    ci = np.zeros((8, HLP), np.int32)
    ci[0] = _WCOL - 1
    ci[1] = _HCOL - 1
    ci[2] = _STARTCOL
    ci[3] = _HMAP
    ci[4] = _WCOL
    cf = np.zeros((8, HLP), np.float32)
    cf[0] = _WCOL.astype(np.float32)
    cf[1] = _HCOL.astype(np.float32)

    val, idx, w = _run_prep(
        q_t, rpx, rpy, W_val, b_val.reshape(1, EMBED), W_off_x, b_off_x,
        W_off_y, b_off_y, W_attn, b_attn.reshape(1, HLP),
        jnp.asarray(ci), jnp.asarray(cf))

    table = val.reshape(BS * NQ * HEADS, HEAD_DIM)
    idx = idx.reshape(N_ITEMS, 4, HLP)
    w = w.reshape(N_ITEMS, 4 * HLP)

    msda = _run_sc(table, idx, w).reshape(BS, NQ, EMBED)

    out_t = _run_out(msda, q_t, W_out, b_out.reshape(1, EMBED))
    return jnp.transpose(out_t, (1, 0, 2))


# speedup vs baseline: 1.0486x; 1.0486x over previous
import functools

import numpy as np

import jax
import jax.numpy as jnp
from jax import lax
from jax.experimental import pallas as pl
from jax.experimental.pallas import tpu as pltpu
from jax.experimental.pallas import tpu_sc as plsc

EMBED = 256
HEADS = 8
LEVELS = 4
POINTS = 4
SHAPES = ((64, 64), (32, 32), (16, 16), (8, 8))
BS = 2
NQ = sum(h * w for h, w in SHAPES)
HEAD_DIM = EMBED // HEADS
HLP = HEADS * LEVELS * POINTS
TQ = 680
NJ = NQ // TQ

_LMAP = np.array([(c // POINTS) % LEVELS for c in range(HLP)], np.int32)
_HMAP = np.array([c // (LEVELS * POINTS) for c in range(HLP)], np.int32)
_STARTS = np.cumsum([0] + [h * w for h, w in SHAPES])[:LEVELS].astype(np.int32)
_WCOL = np.array([SHAPES[l][1] for l in _LMAP], np.int32)
_HCOL = np.array([SHAPES[l][0] for l in _LMAP], np.int32)
_STARTCOL = _STARTS[_LMAP]

N_ITEMS = BS * NQ
N_TILES = 32
ITEMS_PER_TILE = N_ITEMS // N_TILES


def _prep_body(q_ref, rpx_ref, rpy_ref, wval_ref, bval_ref, woffx_ref,
               boffx_ref, woffy_ref, boffy_ref, wattn_ref, battn_ref,
               ci_ref, cf_ref, val_ref, idx_ref, w_ref):
    b = pl.program_id(0)
    q = q_ref[0]

    dot = functools.partial(
        lax.dot_general, dimension_numbers=(((1,), (1,)), ((), ())),
        preferred_element_type=jnp.float32)

    val_ref[0] = (dot(q, wval_ref[...])
                  + bval_ref[0][None, :]).astype(jnp.bfloat16)

    offx = dot(q, woffx_ref[...]) + boffx_ref[0][None, :]
    offy = dot(q, woffy_ref[...]) + boffy_ref[0][None, :]
    logits = dot(q, wattn_ref[...]) + battn_ref[0][None, :]

    z = logits.reshape(TQ, HEADS, LEVELS * POINTS)
    z = z - jnp.max(z, axis=-1, keepdims=True)
    e = jnp.exp(z)
    aw = (e / jnp.sum(e, axis=-1, keepdims=True)).reshape(TQ, HLP)

    wcol = cf_ref[0][None, :]
    hcol = cf_ref[1][None, :]
    wm1 = ci_ref[0][None, :]
    hm1 = ci_ref[1][None, :]
    start = ci_ref[2][None, :]
    hmap = ci_ref[3][None, :]
    wint = ci_ref[4][None, :]

    px = rpx_ref[0] * wcol + offx - 0.5
    py = rpy_ref[0] * hcol + offy - 0.5
    x0f = jnp.floor(px)
    y0f = jnp.floor(py)
    fx = px - x0f
    fy = py - y0f
    ix0 = x0f.astype(jnp.int32)
    iy0 = y0f.astype(jnp.int32)
    wxs = (1.0 - fx, fx)
    wys = (1.0 - fy, fy)

    base_row = b * (NQ * HEADS) + hmap
    for cy in range(2):
        for cx in range(2):
            ci = cy * 2 + cx
            ix = ix0 + cx
            iy = iy0 + cy
            valid = ((ix >= 0) & (ix <= wm1) & (iy >= 0) & (iy <= hm1))
            ixc = jnp.clip(ix, 0, wm1)
            iyc = jnp.clip(iy, 0, hm1)
            cell = start + iyc * wint + ixc
            idx_ref[0, :, ci, :] = base_row + cell * HEADS
            w_ref[0, :, ci, :] = aw * wxs[cx] * wys[cy] * valid.astype(jnp.float32)


def _out_body(m_ref, q_ref, wout_ref, bout_ref, o_ref):
    dot = functools.partial(
        lax.dot_general, dimension_numbers=(((1,), (1,)), ((), ())),
        preferred_element_type=jnp.float32)
    o_ref[0] = dot(m_ref[0], wout_ref[...]) + bout_ref[0][None, :] + q_ref[0]


def _sc_body(table_ref, idx_hbm, w_hbm, out_hbm, idx_v, w_v, rows_v, out_v,
             g_sem, io_sem, o_sem):
    nc = 2
    wid = lax.axis_index("s") * nc + lax.axis_index("c")
    base = wid * ITEMS_PER_TILE

    def fire_io(j, bb):
        pltpu.async_copy(idx_hbm.at[base + j],
                         idx_v.at[pl.ds(bb * 4, 4)], io_sem.at[bb])
        pltpu.async_copy(w_hbm.at[base + j], w_v.at[bb], io_sem.at[bb])

    def wait_io(bb):
        pltpu.make_async_copy(idx_hbm.at[0], idx_v.at[pl.ds(bb * 4, 4)],
                              io_sem.at[bb]).wait()
        pltpu.make_async_copy(w_hbm.at[0], w_v.at[bb], io_sem.at[bb]).wait()

    def fire_gathers(bb):
        for c in range(4):
            pltpu.async_copy(table_ref.at[idx_v.at[bb * 4 + c]],
                             rows_v.at[bb, c], g_sem.at[bb, c])

    def wait_gathers(bb):
        for c in range(4):
            pltpu.make_async_copy(table_ref.at[idx_v.at[bb * 4 + c]],
                                  rows_v.at[bb, c], g_sem.at[bb, c]).wait()

    def wait_out(bb):
        pltpu.make_async_copy(out_v.at[bb], out_hbm.at[0], o_sem.at[bb]).wait()

    fire_io(0, 0)
    fire_io(1, 1)
    wait_io(0)
    fire_gathers(0)

    def pair_body(i, carry):
        j0 = i * 2
        for b in range(2):
            j = j0 + b
            wait_gathers(b)

            @pl.when(j + 1 < ITEMS_PER_TILE)
            def _(b=b):
                wait_io(1 - b)
                fire_gathers(1 - b)

            @pl.when(j >= 2)
            def _(b=b):
                wait_out(b)

            w_slice = w_v.at[b]
            zero = jnp.zeros((16,), jnp.float32)
            for h in range(HEADS):
                parts = []
                for c in range(4):
                    rbase = c * HLP + h * 16

                    def k_body(k0, carry, b=b, c=c, h=h, rbase=rbase):
                        a0e, a0o, a1e, a1o = carry
                        for dk in range(8):
                            kk = k0 * 8 + dk
                            wv = plsc.load_gather(
                                w_slice,
                                [jnp.full((16,), rbase + kk, jnp.int32)])
                            rv = rows_v[b, c, h * 16 + kk, :]
                            r0, r1 = plsc.unpack(
                                rv, format=plsc.PackFormat.INTERLEAVED,
                                preferred_element_type=jnp.float32)
                            if dk % 2 == 0:
                                a0e = a0e + wv * r0
                                a1e = a1e + wv * r1
                            else:
                                a0o = a0o + wv * r0
                                a1o = a1o + wv * r1
                        return a0e, a0o, a1e, a1o

                    a0e, a0o, a1e, a1o = lax.fori_loop(
                        0, 2, k_body, (zero, zero, zero, zero))
                    parts.append((a0e + a0o, a1e + a1o))
                acc0 = (parts[0][0] + parts[1][0]) + (parts[2][0] + parts[3][0])
                acc1 = (parts[0][1] + parts[1][1]) + (parts[2][1] + parts[3][1])
                out_v[b, pl.ds(h * 32, 16)] = acc0
                out_v[b, pl.ds(h * 32 + 16, 16)] = acc1
            pltpu.async_copy(out_v.at[b], out_hbm.at[base + j], o_sem.at[b])

            @pl.when(j + 2 < ITEMS_PER_TILE)
            def _(b=b, j=j):
                fire_io(j + 2, b)
        return carry

    lax.fori_loop(0, ITEMS_PER_TILE // 2, pair_body, 0)
    wait_out(0)
    wait_out(1)


def _run_prep(q_t, rpx, rpy, W_val, b_val, W_off_x, b_off_x, W_off_y,
              b_off_y, W_attn, b_attn, ci, cf, interpret=False):
    grid = (BS, NJ)
    full = lambda shape: pl.BlockSpec(shape, lambda b, j: tuple(0 for _ in shape))
    return pl.pallas_call(
        _prep_body,
        grid=grid,
        in_specs=[
            pl.BlockSpec((1, TQ, EMBED), lambda b, j: (b, j, 0)),
            pl.BlockSpec((1, TQ, HLP), lambda b, j: (b, j, 0)),
            pl.BlockSpec((1, TQ, HLP), lambda b, j: (b, j, 0)),
            full((EMBED, EMBED)),
            full((1, EMBED)),
            full((HLP, EMBED)),
            full((1, HLP)),
            full((HLP, EMBED)),
            full((1, HLP)),
            full((HLP, EMBED)),
            full((1, HLP)),
            full((8, HLP)),
            full((8, HLP)),
        ],
        out_specs=[
            pl.BlockSpec((1, TQ, EMBED), lambda b, j: (b, j, 0)),
            pl.BlockSpec((1, TQ, 4, HLP), lambda b, j: (b, j, 0, 0)),
            pl.BlockSpec((1, TQ, 4, HLP), lambda b, j: (b, j, 0, 0)),
        ],
        out_shape=[
            jax.ShapeDtypeStruct((BS, NQ, EMBED), jnp.bfloat16),
            jax.ShapeDtypeStruct((BS, NQ, 4, HLP), jnp.int32),
            jax.ShapeDtypeStruct((BS, NQ, 4, HLP), jnp.float32),
        ],
        interpret=interpret,
    )(q_t, rpx, rpy, W_val, b_val, W_off_x, b_off_x, W_off_y, b_off_y,
      W_attn, b_attn, ci, cf)


def _run_out(msda, q_t, W_out, b_out, interpret=False):
    full = lambda shape: pl.BlockSpec(shape, lambda b, j: tuple(0 for _ in shape))
    return pl.pallas_call(
        _out_body,
        grid=(BS, NJ),
        in_specs=[
            pl.BlockSpec((1, TQ, EMBED), lambda b, j: (b, j, 0)),
            pl.BlockSpec((1, TQ, EMBED), lambda b, j: (b, j, 0)),
            full((EMBED, EMBED)),
            full((1, EMBED)),
        ],
        out_specs=pl.BlockSpec((1, TQ, EMBED), lambda b, j: (b, j, 0)),
        out_shape=jax.ShapeDtypeStruct((BS, NQ, EMBED), jnp.float32),
        interpret=interpret,
    )(msda, q_t, W_out, b_out)


def _run_sc(table, idx, w):
    mesh = plsc.VectorSubcoreMesh(core_axis_name="c", subcore_axis_name="s")
    kern = functools.partial(
        pl.kernel,
        mesh=mesh,
        out_type=jax.ShapeDtypeStruct((N_ITEMS, EMBED), jnp.float32),
        scratch_types=[
            pltpu.VMEM((8, HLP), jnp.int32),
            pltpu.VMEM((2, 4 * HLP), jnp.float32),
            pltpu.VMEM((2, 4, HLP, HEAD_DIM), jnp.bfloat16),
            pltpu.VMEM((2, EMBED), jnp.float32),
            pltpu.SemaphoreType.DMA((2, 4)),
            pltpu.SemaphoreType.DMA((2,)),
            pltpu.SemaphoreType.DMA((2,)),
        ],
        compiler_params=pltpu.CompilerParams(
            needs_layout_passes=False, use_tc_tiling_on_sc=False),
    )(_sc_body)
    return kern(table, idx, w)


def kernel(query, reference_points, spatial_shapes, W_off, b_off, W_attn,
           b_attn, W_val, b_val, W_out, b_out):
    del spatial_shapes
    q_t = jnp.transpose(query, (1, 0, 2))

    W_off_r = W_off.reshape(HEADS, LEVELS, POINTS, 2, EMBED)
    b_off_r = b_off.reshape(HEADS, LEVELS, POINTS, 2)
    W_off_x = W_off_r[..., 0, :].reshape(HLP, EMBED)
    W_off_y = W_off_r[..., 1, :].reshape(HLP, EMBED)
    b_off_x = b_off_r[..., 0].reshape(1, HLP)
    b_off_y = b_off_r[..., 1].reshape(1, HLP)

    lmap = jnp.asarray(_LMAP)
    rpx = reference_points[..., 0][:, :, lmap]
    rpy = reference_points[..., 1][:, :, lmap]

    ci = np.zeros((8, HLP), np.int32)
    ci[0] = _WCOL - 1
    ci[1] = _HCOL - 1
    ci[2] = _STARTCOL
    ci[3] = _HMAP
    ci[4] = _WCOL
    cf = np.zeros((8, HLP), np.float32)
    cf[0] = _WCOL.astype(np.float32)
    cf[1] = _HCOL.astype(np.float32)

    val, idx, w = _run_prep(
        q_t, rpx, rpy, W_val, b_val.reshape(1, EMBED), W_off_x, b_off_x,
        W_off_y, b_off_y, W_attn, b_attn.reshape(1, HLP),
        jnp.asarray(ci), jnp.asarray(cf))

    table = val.reshape(BS * NQ * HEADS, HEAD_DIM)
    idx = idx.reshape(N_ITEMS, 4, HLP)
    w = w.reshape(N_ITEMS, 4 * HLP)

    msda = _run_sc(table, idx, w).reshape(BS, NQ, EMBED)

    perm = np.concatenate(
        [np.concatenate([h * 32 + 2 * np.arange(16),
                         h * 32 + 2 * np.arange(16) + 1])
         for h in range(HEADS)]).astype(np.int32)
    W_out_use = W_out[:, jnp.asarray(perm)]

    out_t = _run_out(msda, q_t, W_out_use, b_out.reshape(1, EMBED))
    return jnp.transpose(out_t, (1, 0, 2))


# speedup vs baseline: 1.0506x; 1.0019x over previous
import functools

import numpy as np

import jax
import jax.numpy as jnp
from jax import lax
from jax.experimental import pallas as pl
from jax.experimental.pallas import tpu as pltpu
from jax.experimental.pallas import tpu_sc as plsc

EMBED = 256
HEADS = 8
LEVELS = 4
POINTS = 4
SHAPES = ((64, 64), (32, 32), (16, 16), (8, 8))
BS = 2
NQ = sum(h * w for h, w in SHAPES)
HEAD_DIM = EMBED // HEADS
HLP = HEADS * LEVELS * POINTS
TQ = 680
NJ = NQ // TQ

_LMAP = np.array([(c // POINTS) % LEVELS for c in range(HLP)], np.int32)
_HMAP = np.array([c // (LEVELS * POINTS) for c in range(HLP)], np.int32)
_STARTS = np.cumsum([0] + [h * w for h, w in SHAPES])[:LEVELS].astype(np.int32)
_WCOL = np.array([SHAPES[l][1] for l in _LMAP], np.int32)
_HCOL = np.array([SHAPES[l][0] for l in _LMAP], np.int32)
_STARTCOL = _STARTS[_LMAP]

N_ITEMS = BS * NQ
N_TILES = 32
ITEMS_PER_TILE = N_ITEMS // N_TILES


def _prep_body(q_ref, rpx_ref, rpy_ref, wval_ref, bval_ref, woffx_ref,
               boffx_ref, woffy_ref, boffy_ref, wattn_ref, battn_ref,
               ci_ref, cf_ref, val_ref, idx_ref, w_ref):
    b = pl.program_id(0)
    q = q_ref[0]

    dot = functools.partial(
        lax.dot_general, dimension_numbers=(((1,), (1,)), ((), ())),
        preferred_element_type=jnp.float32)

    val_ref[0] = (dot(q, wval_ref[...])
                  + bval_ref[0][None, :]).astype(jnp.bfloat16)

    offx = dot(q, woffx_ref[...]) + boffx_ref[0][None, :]
    offy = dot(q, woffy_ref[...]) + boffy_ref[0][None, :]
    logits = dot(q, wattn_ref[...]) + battn_ref[0][None, :]

    z = logits.reshape(TQ, HEADS, LEVELS * POINTS)
    z = z - jnp.max(z, axis=-1, keepdims=True)
    e = jnp.exp(z)
    aw = (e / jnp.sum(e, axis=-1, keepdims=True)).reshape(TQ, HLP)

    wcol = cf_ref[0][None, :]
    hcol = cf_ref[1][None, :]
    wm1 = ci_ref[0][None, :]
    hm1 = ci_ref[1][None, :]
    start = ci_ref[2][None, :]
    hmap = ci_ref[3][None, :]
    wint = ci_ref[4][None, :]

    px = rpx_ref[0] * wcol + offx - 0.5
    py = rpy_ref[0] * hcol + offy - 0.5
    x0f = jnp.floor(px)
    y0f = jnp.floor(py)
    fx = px - x0f
    fy = py - y0f
    ix0 = x0f.astype(jnp.int32)
    iy0 = y0f.astype(jnp.int32)
    wxs = (1.0 - fx, fx)
    wys = (1.0 - fy, fy)

    base_row = b * (NQ * HEADS) + hmap
    for cy in range(2):
        for cx in range(2):
            ci = cy * 2 + cx
            ix = ix0 + cx
            iy = iy0 + cy
            valid = ((ix >= 0) & (ix <= wm1) & (iy >= 0) & (iy <= hm1))
            ixc = jnp.clip(ix, 0, wm1)
            iyc = jnp.clip(iy, 0, hm1)
            cell = start + iyc * wint + ixc
            idx_ref[0, :, ci, :] = base_row + cell * HEADS
            w_ref[0, :, ci, :] = aw * wxs[cx] * wys[cy] * valid.astype(jnp.float32)


def _out_body(m_ref, q_ref, wout_ref, bout_ref, o_ref):
    dot = functools.partial(
        lax.dot_general, dimension_numbers=(((1,), (1,)), ((), ())),
        preferred_element_type=jnp.float32)
    o_ref[0] = dot(m_ref[0], wout_ref[...]) + bout_ref[0][None, :] + q_ref[0]


def _sc_body(table_ref, idx_hbm, w_hbm, out_hbm, idx_v, w_v, rows_v, out_v,
             g_sem, io_sem, o_sem):
    nc = 2
    wid = lax.axis_index("s") * nc + lax.axis_index("c")
    base = wid * ITEMS_PER_TILE

    def fire_io(j, bb):
        pltpu.async_copy(idx_hbm.at[base + j],
                         idx_v.at[pl.ds(bb * 4, 4)], io_sem.at[bb])
        pltpu.async_copy(w_hbm.at[base + j], w_v.at[bb], io_sem.at[bb])

    def wait_io(bb):
        pltpu.make_async_copy(idx_hbm.at[0], idx_v.at[pl.ds(bb * 4, 4)],
                              io_sem.at[bb]).wait()
        pltpu.make_async_copy(w_hbm.at[0], w_v.at[bb], io_sem.at[bb]).wait()

    def fire_gathers(bb):
        for c in range(4):
            pltpu.async_copy(table_ref.at[idx_v.at[bb * 4 + c]],
                             rows_v.at[bb, c], g_sem.at[bb, c])

    def wait_gathers(bb):
        for c in range(4):
            pltpu.make_async_copy(table_ref.at[idx_v.at[bb * 4 + c]],
                                  rows_v.at[bb, c], g_sem.at[bb, c]).wait()

    def wait_out(bb):
        pltpu.make_async_copy(out_v.at[bb], out_hbm.at[0], o_sem.at[bb]).wait()

    fire_io(0, 0)
    fire_io(1, 1)
    wait_io(0)
    fire_gathers(0)

    def pair_body(i, carry):
        j0 = i * 2
        for b in range(2):
            j = j0 + b
            wait_gathers(b)

            @pl.when(j + 1 < ITEMS_PER_TILE)
            def _(b=b):
                wait_io(1 - b)
                fire_gathers(1 - b)

            @pl.when(j >= 2)
            def _(b=b):
                wait_out(b)

            w_slice = w_v.at[b]
            zero = jnp.zeros((16,), jnp.float32)
            for h in range(HEADS):
                parts = []
                for c in range(4):
                    rbase = c * HLP + h * 16

                    def k_body(k0, carry, b=b, c=c, h=h, rbase=rbase):
                        a0e, a0o, a1e, a1o = carry
                        for dk in range(8):
                            kk = k0 * 8 + dk
                            wv = plsc.load_gather(
                                w_slice,
                                [jnp.full((16,), rbase + kk, jnp.int32)])
                            rv = rows_v[b, c, h * 16 + kk, :]
                            r0, r1 = plsc.unpack(
                                rv, format=plsc.PackFormat.INTERLEAVED,
                                preferred_element_type=jnp.float32)
                            if dk % 2 == 0:
                                a0e = a0e + wv * r0
                                a1e = a1e + wv * r1
                            else:
                                a0o = a0o + wv * r0
                                a1o = a1o + wv * r1
                        return a0e, a0o, a1e, a1o

                    a0e, a0o, a1e, a1o = lax.fori_loop(
                        0, 2, k_body, (zero, zero, zero, zero))
                    parts.append((a0e + a0o, a1e + a1o))
                acc0 = (parts[0][0] + parts[1][0]) + (parts[2][0] + parts[3][0])
                acc1 = (parts[0][1] + parts[1][1]) + (parts[2][1] + parts[3][1])
                out_v[b, pl.ds(h * 32, 16)] = acc0
                out_v[b, pl.ds(h * 32 + 16, 16)] = acc1
            pltpu.async_copy(out_v.at[b], out_hbm.at[base + j], o_sem.at[b])

            @pl.when(j + 2 < ITEMS_PER_TILE)
            def _(b=b, j=j):
                fire_io(j + 2, b)
        return carry

    lax.fori_loop(0, ITEMS_PER_TILE // 2, pair_body, 0)
    wait_out(0)
    wait_out(1)


def _run_prep(q_t, rpx, rpy, W_val, b_val, W_off_x, b_off_x, W_off_y,
              b_off_y, W_attn, b_attn, ci, cf):
    grid = (BS, NJ)
    full = lambda shape: pl.BlockSpec(shape, lambda b, j: tuple(0 for _ in shape))
    return pl.pallas_call(
        _prep_body,
        grid=grid,
        in_specs=[
            pl.BlockSpec((1, TQ, EMBED), lambda b, j: (b, j, 0)),
            pl.BlockSpec((1, TQ, HLP), lambda b, j: (b, j, 0)),
            pl.BlockSpec((1, TQ, HLP), lambda b, j: (b, j, 0)),
            full((EMBED, EMBED)),
            full((1, EMBED)),
            full((HLP, EMBED)),
            full((1, HLP)),
            full((HLP, EMBED)),
            full((1, HLP)),
            full((HLP, EMBED)),
            full((1, HLP)),
            full((8, HLP)),
            full((8, HLP)),
        ],
        out_specs=[
            pl.BlockSpec((1, TQ, EMBED), lambda b, j: (b, j, 0)),
            pl.BlockSpec((1, TQ, 4, HLP), lambda b, j: (b, j, 0, 0)),
            pl.BlockSpec((1, TQ, 4, HLP), lambda b, j: (b, j, 0, 0)),
        ],
        out_shape=[
            jax.ShapeDtypeStruct((BS, NQ, EMBED), jnp.bfloat16),
            jax.ShapeDtypeStruct((BS, NQ, 4, HLP), jnp.int32),
            jax.ShapeDtypeStruct((BS, NQ, 4, HLP), jnp.float32),
        ],
    )(q_t, rpx, rpy, W_val, b_val, W_off_x, b_off_x, W_off_y, b_off_y,
      W_attn, b_attn, ci, cf)


def _run_out(msda, q_t, W_out, b_out):
    full = lambda shape: pl.BlockSpec(shape, lambda b, j: tuple(0 for _ in shape))
    return pl.pallas_call(
        _out_body,
        grid=(BS, NJ),
        in_specs=[
            pl.BlockSpec((1, TQ, EMBED), lambda b, j: (b, j, 0)),
            pl.BlockSpec((1, TQ, EMBED), lambda b, j: (b, j, 0)),
            full((EMBED, EMBED)),
            full((1, EMBED)),
        ],
        out_specs=pl.BlockSpec((1, TQ, EMBED), lambda b, j: (b, j, 0)),
        out_shape=jax.ShapeDtypeStruct((BS, NQ, EMBED), jnp.float32),
    )(msda, q_t, W_out, b_out)


def _run_sc(table, idx, w):
    mesh = plsc.VectorSubcoreMesh(core_axis_name="c", subcore_axis_name="s")
    kern = functools.partial(
        pl.kernel,
        mesh=mesh,
        out_type=jax.ShapeDtypeStruct((N_ITEMS, EMBED), jnp.float32),
        scratch_types=[
            pltpu.VMEM((8, HLP), jnp.int32),
            pltpu.VMEM((2, 4 * HLP), jnp.float32),
            pltpu.VMEM((2, 4, HLP, HEAD_DIM), jnp.bfloat16),
            pltpu.VMEM((2, EMBED), jnp.float32),
            pltpu.SemaphoreType.DMA((2, 4)),
            pltpu.SemaphoreType.DMA((2,)),
            pltpu.SemaphoreType.DMA((2,)),
        ],
        compiler_params=pltpu.CompilerParams(
            needs_layout_passes=False, use_tc_tiling_on_sc=False),
    )(_sc_body)
    return kern(table, idx, w)


def kernel(query, reference_points, spatial_shapes, W_off, b_off, W_attn,
           b_attn, W_val, b_val, W_out, b_out):
    del spatial_shapes
    q_t = jnp.transpose(query, (1, 0, 2))

    W_off_r = W_off.reshape(HEADS, LEVELS, POINTS, 2, EMBED)
    b_off_r = b_off.reshape(HEADS, LEVELS, POINTS, 2)
    W_off_x = W_off_r[..., 0, :].reshape(HLP, EMBED)
    W_off_y = W_off_r[..., 1, :].reshape(HLP, EMBED)
    b_off_x = b_off_r[..., 0].reshape(1, HLP)
    b_off_y = b_off_r[..., 1].reshape(1, HLP)

    lmap = jnp.asarray(_LMAP)
    rpx = reference_points[..., 0][:, :, lmap]
    rpy = reference_points[..., 1][:, :, lmap]

    ci = np.zeros((8, HLP), np.int32)
    ci[0] = _WCOL - 1
    ci[1] = _HCOL - 1
    ci[2] = _STARTCOL
    ci[3] = _HMAP
    ci[4] = _WCOL
    cf = np.zeros((8, HLP), np.float32)
    cf[0] = _WCOL.astype(np.float32)
    cf[1] = _HCOL.astype(np.float32)

    val, idx, w = _run_prep(
        q_t, rpx, rpy, W_val, b_val.reshape(1, EMBED), W_off_x, b_off_x,
        W_off_y, b_off_y, W_attn, b_attn.reshape(1, HLP),
        jnp.asarray(ci), jnp.asarray(cf))

    table = val.reshape(BS * NQ * HEADS, HEAD_DIM)
    idx = idx.reshape(N_ITEMS, 4, HLP)
    w = w.reshape(N_ITEMS, 4 * HLP)

    msda = _run_sc(table, idx, w).reshape(BS, NQ, EMBED)

    perm = np.concatenate(
        [np.concatenate([h * 32 + 2 * np.arange(16),
                         h * 32 + 2 * np.arange(16) + 1])
         for h in range(HEADS)]).astype(np.int32)
    W_out_use = W_out[:, jnp.asarray(perm)]

    out_t = _run_out(msda, q_t, W_out_use, b_out.reshape(1, EMBED))
    return jnp.transpose(out_t, (1, 0, 2))
